# baseline (speedup 1.0000x reference)
import functools

import jax
import jax.numpy as jnp
from jax import lax
from jax.experimental import pallas as pl
from jax.experimental.pallas import tpu as pltpu
from jax.experimental.pallas import tpu_sc as plsc

_N = 2048
_H = 8
_P = 65536
_NC = 2
_NS = 16
_NW = _NC * _NS
_RW = _N // _NW
_WROWS = 32
_NWIN = _RW // _WROWS
_TBL = _WROWS * _N
_CHUNK = 4096
_NSTEP = _CHUNK // 16
_SEGSTEPS = 256
_NSEG = _TBL // (_SEGSTEPS * 16)
_BATCH = _SEGSTEPS * 16
_ZBUF = 16384
_HS = _N * _N


def _proj_body(wg_ref, pe_ref, bg_ref, out_ref):
    out_ref[...] = lax.dot_general(
        wg_ref[...], pe_ref[...],
        dimension_numbers=(((0,), (1,)), ((), ())),
        preferred_element_type=jnp.float32,
    ) + bg_ref[...]


_IOTA = None


def _sc_body(idx_hbm, vals_hbm, out_hbm,
             tbl, zbuf, rowb, colb, ckey, cpid, bkey, bpid,
             gidx, sidx, gval, semz, semg, sems):
    cid = lax.axis_index("c")
    sid = lax.axis_index("s")
    wid = sid * _NC + cid
    row_base = wid * _RW
    iota = lax.iota(jnp.int32, 16)
    zeros16 = jnp.zeros((16,), jnp.float32)
    neg16 = jnp.full((16,), -1, jnp.int32)

    def _zinit(i, carry):
        zbuf[pl.ds(i * 16, 16)] = zeros16
        return carry
    lax.fori_loop(0, _ZBUF // 16, _zinit, 0)

    def _tinit(i, carry):
        tbl[pl.ds(i * 16, 16)] = neg16
        return carry
    lax.fori_loop(0, _TBL // 16, _tinit, 0)

    zcopies = []
    for h in range(_H):
        for k in range(_RW * _N // _ZBUF):
            off = h * _HS + row_base * _N + k * _ZBUF
            zcopies.append(
                pltpu.async_copy(zbuf, out_hbm.at[pl.ds(off, _ZBUF)], semz))

    for win in range(_NWIN):
        lo = row_base + win * _WROWS
        keybase = lo * _N

        def _chunk_body(c, carry, lo=lo, keybase=keybase):
            base = c * _CHUNK
            pltpu.sync_copy(idx_hbm.at[0, pl.ds(base, _CHUNK)], rowb)
            pltpu.sync_copy(idx_hbm.at[1, pl.ds(base, _CHUNK)], colb)

            def _step(s, cnt):
                r16 = rowb[pl.ds(s * 16, 16)] & (_N - 1)
                c16 = colb[pl.ds(s * 16, 16)] & (_N - 1)
                m = (r16 >= lo) & (r16 < lo + _WROWS)
                kl = r16 * _N + c16 - keybase
                pid = base + s * 16 + iota
                npop = plsc.all_reduce_population_count(m)[0]
                spid, skl, _ = plsc.sort_key_val(pid, kl, mask=m)
                ckey[pl.ds(cnt, 16)] = skl
                cpid[pl.ds(cnt, 16)] = spid
                return cnt + npop

            cnt = lax.fori_loop(0, _NSTEP, _step, 0)

            def _ins(s, carry2):
                k16 = ckey[pl.ds(s * 16, 16)]
                p16 = cpid[pl.ds(s * 16, 16)]
                valid = (s * 16 + iota) < cnt
                for lane in range(16):
                    mlane = valid & (iota == lane)
                    plsc.store_scatter(tbl, [k16], p16, mask=mlane)
                return carry2

            lax.fori_loop(0, (cnt + 15) // 16, _ins, 0)
            return carry

        lax.fori_loop(0, _P // _CHUNK, _chunk_body, 0)

        if win == 0:
            for cp in zcopies:
                cp.wait()

        def _seg_body(g, carry, keybase=keybase):
            segbase = g * _SEGSTEPS * 16

            def _estep(t, off):
                s16 = segbase + t * 16
                w16 = tbl[pl.ds(s16, 16)]
                m = w16 >= 0
                kl = s16 + iota
                npop = plsc.all_reduce_population_count(m)[0]
                skl, sw, _ = plsc.sort_key_val(kl, w16, mask=m)
                bkey[pl.ds(off, 16)] = skl
                bpid[pl.ds(off, 16)] = sw
                tbl[pl.ds(s16, 16)] = neg16
                return off + npop

            off = lax.fori_loop(0, _SEGSTEPS, _estep, 0)

            @pl.when(off > 0)
            def _flush():
                nent = off * _H
                nch = (nent + 127) // 128
                nouter = (nch + 7) // 8

                def _outer(o, carry2):
                    gh = []
                    for b in range(8):
                        for s in range(8):
                            e16 = jnp.minimum(
                                (o * 8 + b) * 128 + s * 16 + iota, nent - 1)
                            w16 = e16 >> 3
                            h16 = e16 & 7
                            bk = plsc.load_gather(bkey, [w16])
                            bp = plsc.load_gather(bpid, [w16])
                            gidx[b, pl.ds(s * 16, 16)] = h16 * _P + bp
                            sidx[b, pl.ds(s * 16, 16)] = (
                                h16 * _HS + keybase + bk)
                        gh.append(pltpu.async_copy(
                            vals_hbm.at[gidx.at[b]], gval.at[b], semg))
                    for hnd in gh:
                        hnd.wait()
                    sh = []
                    for b in range(8):
                        sh.append(pltpu.async_copy(
                            gval.at[b], out_hbm.at[sidx.at[b]], sems))
                    for hnd in sh:
                        hnd.wait()
                    return carry2

                lax.fori_loop(0, nouter, _outer, 0)

            return carry

        lax.fori_loop(0, _NSEG, _seg_body, 0)


@functools.partial(
    pl.kernel,
    out_type=jax.ShapeDtypeStruct((_H * _N * _N,), jnp.float32),
    mesh=plsc.VectorSubcoreMesh(
        core_axis_name="c", subcore_axis_name="s",
        num_cores=_NC, num_subcores=_NS),
    compiler_params=pltpu.CompilerParams(needs_layout_passes=False),
    scratch_types=[
        pltpu.VMEM((_TBL,), jnp.int32),
        pltpu.VMEM((_ZBUF,), jnp.float32),
        pltpu.VMEM((_CHUNK,), jnp.int32),
        pltpu.VMEM((_CHUNK,), jnp.int32),
        pltpu.VMEM((_CHUNK + 16,), jnp.int32),
        pltpu.VMEM((_CHUNK + 16,), jnp.int32),
        pltpu.VMEM((_BATCH + 16,), jnp.int32),
        pltpu.VMEM((_BATCH + 16,), jnp.int32),
        pltpu.VMEM((8, 128), jnp.int32),
        pltpu.VMEM((8, 128), jnp.int32),
        pltpu.VMEM((8, 128), jnp.float32),
        pltpu.SemaphoreType.DMA,
        pltpu.SemaphoreType.DMA,
        pltpu.SemaphoreType.DMA,
    ],
)
def _sc_scatter(idx_hbm, vals_hbm, out_hbm, *rest):
    _sc_body(idx_hbm, vals_hbm, out_hbm, *rest)


def kernel(edge_pe_index, edge_pe, num_nodes, W, b, gate):
    P, D = edge_pe.shape
    H = W.shape[1]
    g = jax.nn.sigmoid(gate)
    wg = (W * g[None, :]).astype(jnp.float32)
    bg = (b * g)[:, None].astype(jnp.float32)

    blk = 8192
    vals_t = pl.pallas_call(
        _proj_body,
        out_shape=jax.ShapeDtypeStruct((H, P), jnp.float32),
        grid=(P // blk,),
        in_specs=[
            pl.BlockSpec((D, H), lambda i: (0, 0)),
            pl.BlockSpec((blk, D), lambda i: (i, 0)),
            pl.BlockSpec((H, 1), lambda i: (0, 0)),
        ],
        out_specs=pl.BlockSpec((H, blk), lambda i: (0, i)),
    )(wg, edge_pe, bg)

    vals_flat = vals_t.reshape(H * P)
    idx32 = edge_pe_index.astype(jnp.int32)
    out_flat = _sc_scatter(idx32, vals_flat)
    return out_flat.reshape(_H, _N, _N)


# speedup vs baseline: 2.8361x; 2.8361x over previous
import functools

import jax
import jax.numpy as jnp
from jax import lax
from jax.experimental import pallas as pl
from jax.experimental.pallas import tpu as pltpu
from jax.experimental.pallas import tpu_sc as plsc

_N = 2048
_H = 8
_P = 65536
_NC = 2
_NS = 16
_NW = _NC * _NS
_RW = _N // _NW
_WROWS = 32
_NWIN = _RW // _WROWS
_TBL = _WROWS * _N
_CHUNK = 4096
_NSTEP = _CHUNK // 16
_LCAP = 8192
_BATCH = 4096
_ZBUF = 16384
_HS = _N * _N


def _proj_body(wg_ref, pe_ref, bg_ref, out_ref):
    out_ref[...] = lax.dot_general(
        wg_ref[...], pe_ref[...],
        dimension_numbers=(((0,), (1,)), ((), ())),
        preferred_element_type=jnp.float32,
    ) + bg_ref[...]


_IOTA = None


def _sc_body(idx_hbm, vals_hbm, out_hbm,
             tbl, zbuf, rowb0, colb0, rowb1, colb1, l0, l1, bkey, bpid,
             gidx, sidx, gval, semz, semst, semg, sems):
    cid = lax.axis_index("c")
    sid = lax.axis_index("s")
    wid = sid * _NC + cid
    row_base = wid * _RW
    iota = lax.iota(jnp.int32, 16)
    zeros16 = jnp.zeros((16,), jnp.float32)

    def _zinit(i, carry):
        zbuf[pl.ds(i * 16, 16)] = zeros16
        return carry
    lax.fori_loop(0, _ZBUF // 16, _zinit, 0)

    zcopies = []
    for h in range(_H):
        for k in range(_RW * _N // _ZBUF):
            off = h * _HS + row_base * _N + k * _ZBUF
            zcopies.append(
                pltpu.async_copy(zbuf, out_hbm.at[pl.ds(off, _ZBUF)], semz))

    def _issue(c, bufs):
        base = c * _CHUNK
        return (pltpu.async_copy(idx_hbm.at[0, pl.ds(base, _CHUNK)],
                                 bufs[0], semst),
                pltpu.async_copy(idx_hbm.at[1, pl.ds(base, _CHUNK)],
                                 bufs[1], semst))

    bufs = [(rowb0, colb0), (rowb1, colb1)]
    nchunks = _P // _CHUNK
    pend = _issue(0, bufs[0])
    n0 = jnp.int32(0)
    n1 = jnp.int32(0)
    for c in range(nchunks):
        nxt = _issue(c + 1, bufs[(c + 1) % 2]) if c + 1 < nchunks else None
        for hnd in pend:
            hnd.wait()
        rowb, colb = bufs[c % 2]
        base = c * _CHUNK

        def _step(s, carry, rowb=rowb, colb=colb, base=base):
            a0, a1 = carry
            r16 = rowb[pl.ds(s * 16, 16)] & (_N - 1)
            c16 = colb[pl.ds(s * 16, 16)] & (_N - 1)
            lr = r16 - row_base
            inw = (lr >= 0) & (lr < _RW)
            mw0 = inw & (lr < _WROWS)
            mw1 = inw & (lr >= _WROWS)
            key16 = ((lr & (_WROWS - 1)) << 11) | c16
            packed = ((base + s * 16 + iota) << 16) | key16
            cum0 = plsc.cumsum(jnp.where(mw0, 1, 0))
            pos0 = jnp.minimum(a0 + cum0 - 1, _LCAP - 1)
            plsc.store_scatter(l0, [pos0], packed, mask=mw0)
            cum1 = plsc.cumsum(jnp.where(mw1, 1, 0))
            pos1 = jnp.minimum(a1 + cum1 - 1, _LCAP - 1)
            plsc.store_scatter(l1, [pos1], packed, mask=mw1)
            return (jnp.minimum(a0 + cum0[15], _LCAP),
                    jnp.minimum(a1 + cum1[15], _LCAP))

        n0, n1 = lax.fori_loop(0, _NSTEP, _step, (n0, n1))
        pend = nxt

    for cp in zcopies:
        cp.wait()

    for win, (lst, n) in enumerate(((l0, n0), (l1, n1))):
        keybase = (row_base + win * _WROWS) * _N

        def _ins(s, carry, lst=lst, n=n):
            w16 = lst[pl.ds(s * 16, 16)]
            k16 = w16 & 0xFFFF
            p16 = (w16 >> 16) & 0xFFFF
            valid = (s * 16 + iota) < n
            for lane in range(16):
                plsc.store_scatter(tbl, [k16], p16,
                                   mask=valid & (iota == lane))
            return carry

        lax.fori_loop(0, (n + 15) // 16, _ins, 0)

        def _seg_body(g, carry, lst=lst, n=n, keybase=keybase):
            segstart = g * _BATCH

            def _cstep(t, off):
                e = segstart + t * 16
                w16 = lst[pl.ds(e, 16)]
                k16 = w16 & 0xFFFF
                p16 = (w16 >> 16) & 0xFFFF
                win16 = plsc.load_gather(tbl, [k16])
                m = ((e + iota) < n) & (win16 == p16)
                cum = plsc.cumsum(jnp.where(m, 1, 0))
                pos = off + cum - 1
                plsc.store_scatter(bkey, [pos], k16, mask=m)
                plsc.store_scatter(bpid, [pos], p16, mask=m)
                return off + cum[15]

            rem = jnp.clip(n - segstart, 0, _BATCH)
            off = lax.fori_loop(0, (rem + 15) // 16, _cstep, 0)

            @pl.when(off > 0)
            def _flush():
                nent = off * _H
                nch = (nent + 127) // 128
                nouter = (nch + 7) // 8

                def _outer(o, carry2):
                    gh = []
                    for b in range(8):
                        for s in range(8):
                            e16 = jnp.minimum(
                                (o * 8 + b) * 128 + s * 16 + iota, nent - 1)
                            w16 = e16 >> 3
                            h16 = e16 & 7
                            bk = plsc.load_gather(bkey, [w16])
                            bp = plsc.load_gather(bpid, [w16])
                            gidx[b, pl.ds(s * 16, 16)] = h16 * _P + bp
                            sidx[b, pl.ds(s * 16, 16)] = (
                                h16 * _HS + keybase + bk)
                        gh.append(pltpu.async_copy(
                            vals_hbm.at[gidx.at[b]], gval.at[b], semg))
                    for hnd in gh:
                        hnd.wait()
                    sh = []
                    for b in range(8):
                        sh.append(pltpu.async_copy(
                            gval.at[b], out_hbm.at[sidx.at[b]], sems))
                    for hnd in sh:
                        hnd.wait()
                    return carry2

                lax.fori_loop(0, nouter, _outer, 0)

            return carry

        lax.fori_loop(0, _LCAP // _BATCH, _seg_body, 0)


@functools.partial(
    pl.kernel,
    out_type=jax.ShapeDtypeStruct((_H * _N * _N,), jnp.float32),
    mesh=plsc.VectorSubcoreMesh(
        core_axis_name="c", subcore_axis_name="s",
        num_cores=_NC, num_subcores=_NS),
    compiler_params=pltpu.CompilerParams(needs_layout_passes=False),
    scratch_types=[
        pltpu.VMEM((_TBL,), jnp.int32),
        pltpu.VMEM((_ZBUF,), jnp.float32),
        pltpu.VMEM((_CHUNK,), jnp.int32),
        pltpu.VMEM((_CHUNK,), jnp.int32),
        pltpu.VMEM((_CHUNK,), jnp.int32),
        pltpu.VMEM((_CHUNK,), jnp.int32),
        pltpu.VMEM((_LCAP + 16,), jnp.int32),
        pltpu.VMEM((_LCAP + 16,), jnp.int32),
        pltpu.VMEM((_BATCH + 16,), jnp.int32),
        pltpu.VMEM((_BATCH + 16,), jnp.int32),
        pltpu.VMEM((8, 128), jnp.int32),
        pltpu.VMEM((8, 128), jnp.int32),
        pltpu.VMEM((8, 128), jnp.float32),
        pltpu.SemaphoreType.DMA,
        pltpu.SemaphoreType.DMA,
        pltpu.SemaphoreType.DMA,
        pltpu.SemaphoreType.DMA,
    ],
)
def _sc_scatter(idx_hbm, vals_hbm, out_hbm, *rest):
    _sc_body(idx_hbm, vals_hbm, out_hbm, *rest)


def kernel(edge_pe_index, edge_pe, num_nodes, W, b, gate):
    P, D = edge_pe.shape
    H = W.shape[1]
    g = jax.nn.sigmoid(gate)
    wg = (W * g[None, :]).astype(jnp.float32)
    bg = (b * g)[:, None].astype(jnp.float32)

    blk = 8192
    vals_t = pl.pallas_call(
        _proj_body,
        out_shape=jax.ShapeDtypeStruct((H, P), jnp.float32),
        grid=(P // blk,),
        in_specs=[
            pl.BlockSpec((D, H), lambda i: (0, 0)),
            pl.BlockSpec((blk, D), lambda i: (i, 0)),
            pl.BlockSpec((H, 1), lambda i: (0, 0)),
        ],
        out_specs=pl.BlockSpec((H, blk), lambda i: (0, i)),
    )(wg, edge_pe, bg)

    vals_flat = vals_t.reshape(H * P)
    idx32 = edge_pe_index.astype(jnp.int32)
    out_flat = _sc_scatter(idx32, vals_flat)
    return out_flat.reshape(_H, _N, _N)
